# initial kernel scaffold (unmeasured)
import jax
import jax.numpy as jnp
from jax import lax
from jax.experimental import pallas as pl
from jax.experimental.pallas import tpu as pltpu

N_DEV = 4


def kernel(x, w_mat):
    m_per, k = x.shape
    _, n = w_mat.shape
    n_per = n // N_DEV

    def body(x_ref, w_ref, out_ref, send_buf, recv_buf, send_sems, recv_sems):
        my = lax.axis_index("i")
        x_bf = x_ref[...].astype(jnp.bfloat16)

        rdmas = []
        for d in range(1, N_DEV):
            tgt = lax.rem(my + d, N_DEV)
            w_blk = w_ref[:, pl.ds(tgt * n_per, n_per)].astype(jnp.bfloat16)
            y = jnp.dot(x_bf, w_blk, preferred_element_type=jnp.float32)
            y = y * jax.nn.sigmoid(y)
            send_buf[d - 1] = y.astype(jnp.bfloat16)
            rdma = pltpu.make_async_remote_copy(
                src_ref=send_buf.at[d - 1],
                dst_ref=recv_buf.at[d - 1],
                send_sem=send_sems.at[d - 1],
                recv_sem=recv_sems.at[d - 1],
                device_id=(tgt,),
                device_id_type=pl.DeviceIdType.MESH,
            )
            rdma.start()
            rdmas.append(rdma)

        w_blk = w_ref[:, pl.ds(my * n_per, n_per)].astype(jnp.bfloat16)
        y = jnp.dot(x_bf, w_blk, preferred_element_type=jnp.float32)
        out_ref[pl.ds(my * m_per, m_per), :] = y * jax.nn.sigmoid(y)

        for d in range(1, N_DEV):
            rdmas[d - 1].wait_recv()
            src = lax.rem(my - d + N_DEV, N_DEV)
            out_ref[pl.ds(src * m_per, m_per), :] = (
                recv_buf[d - 1].astype(jnp.float32)
            )

        for rdma in rdmas:
            rdma.wait_send()

    return pl.pallas_call(
        body,
        out_shape=jax.ShapeDtypeStruct((N_DEV * m_per, n_per), jnp.float32),
        in_specs=[
            pl.BlockSpec(memory_space=pltpu.VMEM),
            pl.BlockSpec(memory_space=pltpu.VMEM),
        ],
        out_specs=pl.BlockSpec(memory_space=pltpu.VMEM),
        scratch_shapes=[
            pltpu.VMEM((N_DEV - 1, m_per, n_per), jnp.bfloat16),
            pltpu.VMEM((N_DEV - 1, m_per, n_per), jnp.bfloat16),
            pltpu.SemaphoreType.DMA((N_DEV - 1,)),
            pltpu.SemaphoreType.DMA((N_DEV - 1,)),
        ],
        compiler_params=pltpu.CompilerParams(collective_id=0),
    )(x, w_mat)


# baseline (device time: 32596 ns/iter reference)
import jax
import jax.numpy as jnp
from jax import lax
from jax.experimental import pallas as pl
from jax.experimental.pallas import tpu as pltpu

N_DEV = 4


def kernel(x, w_mat):
    m_per, k = x.shape
    _, n = w_mat.shape
    n_per = n // N_DEV

    def body(x_ref, w_ref, out_ref, send_buf, recv_buf, send_sems, recv_sems):
        my = lax.axis_index("i")
        x_bf = x_ref[...].astype(jnp.bfloat16)

        rdmas = []
        for d in range(1, N_DEV):
            tgt = lax.rem(my + d, N_DEV)
            w_blk = w_ref[:, pl.ds(tgt * n_per, n_per)].astype(jnp.bfloat16)
            y = jnp.dot(x_bf, w_blk, preferred_element_type=jnp.float32)
            y = y * jax.nn.sigmoid(y)
            send_buf[d - 1] = y.astype(jnp.bfloat16)
            rdma = pltpu.make_async_remote_copy(
                src_ref=send_buf.at[d - 1],
                dst_ref=recv_buf.at[d - 1],
                send_sem=send_sems.at[d - 1],
                recv_sem=recv_sems.at[d - 1],
                device_id=(tgt,),
                device_id_type=pl.DeviceIdType.MESH,
            )
            rdma.start()
            rdmas.append(rdma)

        w_blk = w_ref[:, pl.ds(my * n_per, n_per)].astype(jnp.bfloat16)
        y = jnp.dot(x_bf, w_blk, preferred_element_type=jnp.float32)
        out_ref[pl.ds(my * m_per, m_per), :] = y * jax.nn.sigmoid(y)

        for d in range(1, N_DEV):
            rdmas[d - 1].wait_recv()
            src = lax.rem(my - d + N_DEV, N_DEV)
            out_ref[pl.ds(src * m_per, m_per), :] = (
                recv_buf[d - 1].astype(jnp.float32)
            )

        for rdma in rdmas:
            rdma.wait_send()

    return pl.pallas_call(
        body,
        out_shape=jax.ShapeDtypeStruct((N_DEV * m_per, n_per), jnp.float32),
        in_specs=[
            pl.BlockSpec(memory_space=pltpu.VMEM),
            pl.BlockSpec(memory_space=pltpu.VMEM),
        ],
        out_specs=pl.BlockSpec(memory_space=pltpu.VMEM),
        scratch_shapes=[
            pltpu.VMEM((N_DEV - 1, m_per, n_per), jnp.bfloat16),
            pltpu.VMEM((N_DEV - 1, m_per, n_per), jnp.bfloat16),
            pltpu.SemaphoreType.DMA((N_DEV - 1,)),
            pltpu.SemaphoreType.DMA((N_DEV - 1,)),
        ],
    )(x, w_mat)


# device time: 25785 ns/iter; 1.2641x vs baseline; 1.2641x over previous
import jax
import jax.numpy as jnp
from jax import lax
from jax.experimental import pallas as pl
from jax.experimental.pallas import tpu as pltpu

N_DEV = 4


def kernel(x, w_mat):
    m_per, k = x.shape
    _, n = w_mat.shape
    n_per = n // N_DEV

    def body(
        x_hbm,
        w_hbm,
        out_ref,
        xv,
        wv,
        send_buf,
        recv_buf,
        x_sem,
        w_sems,
        send_sems,
        recv_sems,
    ):
        my = lax.axis_index("i")
        targets = [lax.rem(my + d, N_DEV) for d in range(1, N_DEV)] + [my]

        x_dma = pltpu.make_async_copy(x_hbm, xv, x_sem)
        x_dma.start()

        def w_dma(s):
            return pltpu.make_async_copy(
                w_hbm.at[:, pl.ds(targets[s] * n_per, n_per)],
                wv.at[s % 2],
                w_sems.at[s % 2],
            )

        w_dmas = [w_dma(0), w_dma(1)]
        w_dmas[0].start()
        w_dmas[1].start()

        barrier = pltpu.get_barrier_semaphore()
        for d in range(1, N_DEV):
            pl.semaphore_signal(
                barrier,
                inc=1,
                device_id=(targets[d - 1],),
                device_id_type=pl.DeviceIdType.MESH,
            )
        pl.semaphore_wait(barrier, N_DEV - 1)

        x_dma.wait()
        x_bf = xv[...].astype(jnp.bfloat16)

        rdmas = []
        for s in range(N_DEV):
            w_dmas[s].wait()
            y = jnp.dot(
                x_bf,
                wv[s % 2].astype(jnp.bfloat16),
                preferred_element_type=jnp.float32,
            )
            y = y * jax.nn.sigmoid(y)
            if s < N_DEV - 1:
                send_buf[s] = y.astype(jnp.bfloat16)
                rdma = pltpu.make_async_remote_copy(
                    src_ref=send_buf.at[s],
                    dst_ref=recv_buf.at[s],
                    send_sem=send_sems.at[s],
                    recv_sem=recv_sems.at[s],
                    device_id=(targets[s],),
                    device_id_type=pl.DeviceIdType.MESH,
                )
                rdma.start()
                rdmas.append(rdma)
                if s + 2 < N_DEV:
                    w_dmas.append(w_dma(s + 2))
                    w_dmas[s + 2].start()
            else:
                out_ref[pl.ds(my * m_per, m_per), :] = y

        for d in range(1, N_DEV):
            rdmas[d - 1].wait_recv()
            src = lax.rem(my - d + N_DEV, N_DEV)
            out_ref[pl.ds(src * m_per, m_per), :] = (
                recv_buf[d - 1].astype(jnp.float32)
            )

        for rdma in rdmas:
            rdma.wait_send()

    return pl.pallas_call(
        body,
        out_shape=jax.ShapeDtypeStruct((N_DEV * m_per, n_per), jnp.float32),
        in_specs=[
            pl.BlockSpec(memory_space=pl.ANY),
            pl.BlockSpec(memory_space=pl.ANY),
        ],
        out_specs=pl.BlockSpec(memory_space=pltpu.VMEM),
        scratch_shapes=[
            pltpu.VMEM((m_per, k), jnp.float32),
            pltpu.VMEM((2, k, n_per), jnp.float32),
            pltpu.VMEM((N_DEV - 1, m_per, n_per), jnp.bfloat16),
            pltpu.VMEM((N_DEV - 1, m_per, n_per), jnp.bfloat16),
            pltpu.SemaphoreType.DMA,
            pltpu.SemaphoreType.DMA((2,)),
            pltpu.SemaphoreType.DMA((N_DEV - 1,)),
            pltpu.SemaphoreType.DMA((N_DEV - 1,)),
        ],
        compiler_params=pltpu.CompilerParams(collective_id=0),
    )(x, w_mat)


# device time: 24923 ns/iter; 1.3079x vs baseline; 1.0346x over previous
import jax
import jax.numpy as jnp
from jax import lax
from jax.experimental import pallas as pl
from jax.experimental.pallas import tpu as pltpu

N_DEV = 4

_ORDER = [(1, 0), (3, 0), (2, 0), (1, 1), (3, 1), (2, 1), (0, 0), (0, 1)]
_DMA_DEPTH = 3


def kernel(x, w_mat):
    m_per, k = x.shape
    _, n = w_mat.shape
    n_per = n // N_DEV
    n_half = n_per // 2

    def body(
        x_hbm,
        w_hbm,
        out_hbm,
        xv,
        wv,
        send_buf,
        recv_buf,
        stage,
        credit_sems,
        x_sem,
        w_sems,
        send_sems,
        recv_sems,
        out_sems,
    ):
        my = lax.axis_index("i")

        barrier = pltpu.get_barrier_semaphore()
        for d in range(1, N_DEV):
            tgt = lax.rem(my + d, N_DEV)
            pl.semaphore_signal(
                credit_sems.at[N_DEV - 1 - d],
                inc=1,
                device_id=(tgt,),
                device_id_type=pl.DeviceIdType.MESH,
            )
            pl.semaphore_signal(
                barrier,
                inc=1,
                device_id=(tgt,),
                device_id_type=pl.DeviceIdType.MESH,
            )

        x_dma = pltpu.make_async_copy(x_hbm, xv, x_sem)
        x_dma.start()

        def w_dma(c):
            d, h = _ORDER[c]
            tgt = lax.rem(my + d, N_DEV)
            return pltpu.make_async_copy(
                w_hbm.at[:, pl.ds(tgt * n_per + h * n_half, n_half)],
                wv.at[c % _DMA_DEPTH],
                w_sems.at[c % _DMA_DEPTH],
            )

        w_dmas = [w_dma(c) for c in range(_DMA_DEPTH)]
        for dma in w_dmas:
            dma.start()

        x_dma.wait()
        x_bf = xv[...].astype(jnp.bfloat16)

        def silu(v):
            return v * jax.nn.sigmoid(v)

        out_dmas = []

        def flush_block(row_blk, blk_slot):
            dma = pltpu.make_async_copy(
                stage.at[blk_slot],
                out_hbm.at[pl.ds(row_blk * m_per, m_per), :],
                out_sems.at[blk_slot],
            )
            dma.start()
            out_dmas.append(dma)

        rdmas = {}
        for c, (d, h) in enumerate(_ORDER):
            w_dmas[c].wait()
            y = jnp.dot(
                x_bf,
                wv[c % _DMA_DEPTH].astype(jnp.bfloat16),
                preferred_element_type=jnp.float32,
            )
            if c + _DMA_DEPTH < len(_ORDER):
                w_dmas.append(w_dma(c + _DMA_DEPTH))
                w_dmas[c + _DMA_DEPTH].start()
            if d == 0:
                stage[N_DEV - 1, :, pl.ds(h * n_half, n_half)] = silu(y)
                if h == 1:
                    flush_block(my, N_DEV - 1)
            else:
                slot = (d - 1) * 2 + h
                send_buf[slot] = y.astype(jnp.bfloat16)
                if h == 0:
                    pl.semaphore_wait(credit_sems.at[d - 1], 1)
                rdma = pltpu.make_async_remote_copy(
                    src_ref=send_buf.at[slot],
                    dst_ref=recv_buf.at[slot],
                    send_sem=send_sems.at[slot],
                    recv_sem=recv_sems.at[slot],
                    device_id=(lax.rem(my + d, N_DEV),),
                    device_id_type=pl.DeviceIdType.MESH,
                )
                rdma.start()
                rdmas[slot] = rdma

        for d, h in _ORDER[: 2 * (N_DEV - 1)]:
            slot = (d - 1) * 2 + h
            rdmas[slot].wait_recv()
            src = lax.rem(my - d + N_DEV, N_DEV)
            stage[d - 1, :, pl.ds(h * n_half, n_half)] = silu(
                recv_buf[slot].astype(jnp.float32)
            )
            if h == 1:
                flush_block(src, d - 1)

        for rdma in rdmas.values():
            rdma.wait_send()
        for dma in out_dmas:
            dma.wait()
        pl.semaphore_wait(barrier, N_DEV - 1)

    return pl.pallas_call(
        body,
        out_shape=jax.ShapeDtypeStruct((N_DEV * m_per, n_per), jnp.float32),
        in_specs=[
            pl.BlockSpec(memory_space=pltpu.MemorySpace.HBM),
            pl.BlockSpec(memory_space=pltpu.MemorySpace.HBM),
        ],
        out_specs=pl.BlockSpec(memory_space=pltpu.MemorySpace.HBM),
        scratch_shapes=[
            pltpu.VMEM((m_per, k), jnp.float32),
            pltpu.VMEM((_DMA_DEPTH, k, n_half), jnp.float32),
            pltpu.VMEM((2 * (N_DEV - 1), m_per, n_half), jnp.bfloat16),
            pltpu.VMEM((2 * (N_DEV - 1), m_per, n_half), jnp.bfloat16),
            pltpu.VMEM((N_DEV, m_per, n_per), jnp.float32),
            pltpu.SemaphoreType.REGULAR((N_DEV - 1,)),
            pltpu.SemaphoreType.DMA,
            pltpu.SemaphoreType.DMA((_DMA_DEPTH,)),
            pltpu.SemaphoreType.DMA((2 * (N_DEV - 1),)),
            pltpu.SemaphoreType.DMA((2 * (N_DEV - 1),)),
            pltpu.SemaphoreType.DMA((N_DEV,)),
        ],
        compiler_params=pltpu.CompilerParams(collective_id=0),
    )(x, w_mat)
